# baseline (speedup 1.0000x reference)
import functools

import jax
import jax.numpy as jnp
from jax.experimental import pallas as pl
from jax.experimental.pallas import tpu as pltpu

_TOPK = 10
_KCAP = 16


def _fused_topk_kernel(nkeys, bk, nblk,
                       qt_ref, nq_ref, kb_ref, nk_ref,
                       vals_ref, idx_ref,
                       s_ref, v_ref, i_ref):
    blk = pl.program_id(0)
    nq_cols = qt_ref.shape[1]

    @pl.when(blk == 0)
    def _init():
        v_ref[...] = jnp.full((_KCAP, nq_cols), -jnp.inf, jnp.float32)
        i_ref[...] = jnp.zeros((_KCAP, nq_cols), jnp.int32)

    dots = jax.lax.dot_general(
        kb_ref[...], qt_ref[...], (((1,), (0,)), ((), ())),
        preferred_element_type=jnp.float32)
    denom = nk_ref[...] * nq_ref[...] + 1e-8
    s_ref[...] = dots / denom

    @pl.when(blk == nblk - 1)
    def _mask_pad():
        row_ids = jax.lax.broadcasted_iota(jnp.int32, (bk, nq_cols), 0)
        s_ref[...] = jnp.where(row_ids + blk * bk < nkeys,
                               s_ref[...], -jnp.inf)

    qs = min(256, nq_cols)
    for g in range(nq_cols // qs):
        q0 = g * qs

        def cond(carry):
            return carry[0]

        def body(carry, q0=q0):
            _, m = carry
            s = s_ref[:, q0:q0 + qs]
            mi = jnp.argmax(s, axis=0)[None, :].astype(jnp.int32)
            v = v_ref[:, q0:q0 + qs]
            iv = i_ref[:, q0:q0 + qs]
            tmin = v[_TOPK - 1:_TOPK, :]
            upd = m > tmin
            rows = jax.lax.broadcasted_iota(jnp.int32, (bk, qs), 0)
            s_next = jnp.where(rows == mi, -jnp.inf, s)
            s_ref[:, q0:q0 + qs] = s_next
            pos = jnp.sum((v >= m).astype(jnp.int32), axis=0,
                          keepdims=True)
            krows = jax.lax.broadcasted_iota(jnp.int32, (_KCAP, qs), 0)
            v_shift = jnp.concatenate([v[:1], v[:_KCAP - 1]], axis=0)
            i_shift = jnp.concatenate([iv[:1], iv[:_KCAP - 1]], axis=0)
            gidx = mi + blk * bk
            newv = jnp.where(krows < pos, v,
                             jnp.where(krows == pos, m, v_shift))
            newi = jnp.where(krows < pos, iv,
                             jnp.where(krows == pos, gidx, i_shift))
            v_ref[:, q0:q0 + qs] = jnp.where(upd, newv, v)
            i_ref[:, q0:q0 + qs] = jnp.where(upd, newi, iv)
            m2 = jnp.max(s_next, axis=0, keepdims=True)
            cont = jnp.any(m2 > v_ref[_TOPK - 1:_TOPK, q0:q0 + qs])
            return cont, m2

        m0 = jnp.max(s_ref[:, q0:q0 + qs], axis=0, keepdims=True)
        cont0 = jnp.any(m0 > v_ref[_TOPK - 1:_TOPK, q0:q0 + qs])
        jax.lax.while_loop(cond, body, (cont0, m0))

    @pl.when(blk == nblk - 1)
    def _emit():
        vals_ref[...] = v_ref[:_TOPK, :]
        idx_ref[...] = i_ref[:_TOPK, :]


def kernel(queries, keys, k):
    del k
    q, d = queries.shape
    nkeys = keys.shape[0]
    bk = 1024
    nblk = (nkeys + bk - 1) // bk
    kpad = nblk * bk

    norms_text = jnp.linalg.norm(keys, axis=-1)
    norm_question = jnp.linalg.norm(queries, axis=-1)

    qt = queries.T
    keys_pad = jnp.pad(keys, ((0, kpad - nkeys), (0, 0)))
    nk = jnp.pad(norms_text, (0, kpad - nkeys))[:, None]
    nq = norm_question[None, :]

    body = functools.partial(_fused_topk_kernel, nkeys, bk, nblk)
    vals_t, idx_t = pl.pallas_call(
        body,
        grid=(nblk,),
        in_specs=[
            pl.BlockSpec((d, q), lambda i: (0, 0)),
            pl.BlockSpec((1, q), lambda i: (0, 0)),
            pl.BlockSpec((bk, d), lambda i: (i, 0)),
            pl.BlockSpec((bk, 1), lambda i: (i, 0)),
        ],
        out_specs=[
            pl.BlockSpec((_TOPK, q), lambda i: (0, 0)),
            pl.BlockSpec((_TOPK, q), lambda i: (0, 0)),
        ],
        out_shape=[
            jax.ShapeDtypeStruct((_TOPK, q), jnp.float32),
            jax.ShapeDtypeStruct((_TOPK, q), jnp.int32),
        ],
        scratch_shapes=[
            pltpu.VMEM((bk, q), jnp.float32),
            pltpu.VMEM((_KCAP, q), jnp.float32),
            pltpu.VMEM((_KCAP, q), jnp.int32),
        ],
        compiler_params=pltpu.CompilerParams(
            dimension_semantics=("arbitrary",)),
    )(qt, nq, keys_pad, nk)

    return vals_t.T, idx_t.T


# speedup vs baseline: 1.1926x; 1.1926x over previous
import functools

import jax
import jax.numpy as jnp
from jax.experimental import pallas as pl
from jax.experimental.pallas import tpu as pltpu

_TOPK = 10
_KCAP = 16


def _fused_topk_kernel(nkeys, bk, nblk,
                       qt_ref, nq_ref, kb_ref, nk_ref,
                       vals_ref, idx_ref,
                       s_ref, v_ref, i_ref):
    blk = pl.program_id(0)
    nq_cols = qt_ref.shape[1]

    @pl.when(blk == 0)
    def _init():
        v_ref[...] = jnp.full((_KCAP, nq_cols), -jnp.inf, jnp.float32)
        i_ref[...] = jnp.zeros((_KCAP, nq_cols), jnp.int32)

    dots = jax.lax.dot_general(
        kb_ref[...], qt_ref[...], (((1,), (0,)), ((), ())),
        preferred_element_type=jnp.float32)
    denom = nk_ref[...] * nq_ref[...] + 1e-8
    s_ref[...] = dots / denom

    @pl.when(blk == nblk - 1)
    def _mask_pad():
        row_ids = jax.lax.broadcasted_iota(jnp.int32, (bk, nq_cols), 0)
        s_ref[...] = jnp.where(row_ids + blk * bk < nkeys,
                               s_ref[...], -jnp.inf)

    def _extract_one(m):
        s = s_ref[...]
        mi = jnp.argmax(s, axis=0)[None, :].astype(jnp.int32)
        v = v_ref[...]
        iv = i_ref[...]
        tmin = v[_TOPK - 1:_TOPK, :]
        upd = m > tmin
        rows = jax.lax.broadcasted_iota(jnp.int32, (bk, nq_cols), 0)
        s_next = jnp.where(rows == mi, -jnp.inf, s)
        s_ref[...] = s_next
        pos = jnp.sum((v >= m).astype(jnp.int32), axis=0, keepdims=True)
        krows = jax.lax.broadcasted_iota(jnp.int32, (_KCAP, nq_cols), 0)
        v_shift = jnp.concatenate([v[:1], v[:_KCAP - 1]], axis=0)
        i_shift = jnp.concatenate([iv[:1], iv[:_KCAP - 1]], axis=0)
        gidx = mi + blk * bk
        newv = jnp.where(krows < pos, v,
                         jnp.where(krows == pos, m, v_shift))
        newi = jnp.where(krows < pos, iv,
                         jnp.where(krows == pos, gidx, i_shift))
        v_ref[...] = jnp.where(upd, newv, v)
        i_ref[...] = jnp.where(upd, newi, iv)
        return jnp.max(s_next, axis=0, keepdims=True)

    def cond(carry):
        return carry[0]

    def body(carry):
        _, m = carry
        m2 = _extract_one(m)
        m3 = _extract_one(m2)
        cont = jnp.any(m3 > v_ref[_TOPK - 1:_TOPK, :])
        return cont, m3

    m0 = jnp.max(s_ref[...], axis=0, keepdims=True)
    cont0 = jnp.any(m0 > v_ref[_TOPK - 1:_TOPK, :])
    jax.lax.while_loop(cond, body, (cont0, m0))

    @pl.when(blk == nblk - 1)
    def _emit():
        vals_ref[...] = v_ref[:_TOPK, :]
        idx_ref[...] = i_ref[:_TOPK, :]


def kernel(queries, keys, k):
    del k
    q, d = queries.shape
    nkeys = keys.shape[0]
    bk = 1024
    nblk = (nkeys + bk - 1) // bk
    kpad = nblk * bk

    norms_text = jnp.linalg.norm(keys, axis=-1)
    norm_question = jnp.linalg.norm(queries, axis=-1)

    qt = queries.T
    keys_pad = jnp.pad(keys, ((0, kpad - nkeys), (0, 0)))
    nk = jnp.pad(norms_text, (0, kpad - nkeys))[:, None]
    nq = norm_question[None, :]

    body = functools.partial(_fused_topk_kernel, nkeys, bk, nblk)
    vals_t, idx_t = pl.pallas_call(
        body,
        grid=(nblk,),
        in_specs=[
            pl.BlockSpec((d, q), lambda i: (0, 0)),
            pl.BlockSpec((1, q), lambda i: (0, 0)),
            pl.BlockSpec((bk, d), lambda i: (i, 0)),
            pl.BlockSpec((bk, 1), lambda i: (i, 0)),
        ],
        out_specs=[
            pl.BlockSpec((_TOPK, q), lambda i: (0, 0)),
            pl.BlockSpec((_TOPK, q), lambda i: (0, 0)),
        ],
        out_shape=[
            jax.ShapeDtypeStruct((_TOPK, q), jnp.float32),
            jax.ShapeDtypeStruct((_TOPK, q), jnp.int32),
        ],
        scratch_shapes=[
            pltpu.VMEM((bk, q), jnp.float32),
            pltpu.VMEM((_KCAP, q), jnp.float32),
            pltpu.VMEM((_KCAP, q), jnp.int32),
        ],
        compiler_params=pltpu.CompilerParams(
            dimension_semantics=("arbitrary",)),
    )(qt, nq, keys_pad, nk)

    return vals_t.T, idx_t.T


# speedup vs baseline: 1.2241x; 1.0264x over previous
import functools

import jax
import jax.numpy as jnp
from jax.experimental import pallas as pl
from jax.experimental.pallas import tpu as pltpu

_TOPK = 10
_KCAP = 16


def _fused_topk_kernel(nkeys, bk, nblk,
                       qt_ref, nq_ref, kb_ref, nk_ref,
                       vals_ref, idx_ref,
                       s_ref, v_ref, i_ref):
    blk = pl.program_id(0)
    nq_cols = qt_ref.shape[1]

    @pl.when(blk == 0)
    def _init():
        v_ref[...] = jnp.full((_KCAP, nq_cols), -jnp.inf, jnp.float32)
        i_ref[...] = jnp.zeros((_KCAP, nq_cols), jnp.int32)

    dots = jax.lax.dot_general(
        kb_ref[...], qt_ref[...], (((1,), (0,)), ((), ())),
        preferred_element_type=jnp.float32)
    denom = nk_ref[...] * nq_ref[...] + 1e-8
    s_ref[...] = dots / denom

    @pl.when(blk == nblk - 1)
    def _mask_pad():
        row_ids = jax.lax.broadcasted_iota(jnp.int32, (bk, nq_cols), 0)
        s_ref[...] = jnp.where(row_ids + blk * bk < nkeys,
                               s_ref[...], -jnp.inf)

    def _extract_one(m):
        s = s_ref[...]
        mi = jnp.argmax(s, axis=0)[None, :].astype(jnp.int32)
        v = v_ref[...]
        iv = i_ref[...]
        tmin = v[_TOPK - 1:_TOPK, :]
        upd = m > tmin
        rows = jax.lax.broadcasted_iota(jnp.int32, (bk, nq_cols), 0)
        s_next = jnp.where(rows == mi, -jnp.inf, s)
        s_ref[...] = s_next
        pos = jnp.sum((v >= m).astype(jnp.int32), axis=0, keepdims=True)
        krows = jax.lax.broadcasted_iota(jnp.int32, (_KCAP, nq_cols), 0)
        v_shift = jnp.concatenate([v[:1], v[:_KCAP - 1]], axis=0)
        i_shift = jnp.concatenate([iv[:1], iv[:_KCAP - 1]], axis=0)
        gidx = mi + blk * bk
        newv = jnp.where(krows < pos, v,
                         jnp.where(krows == pos, m, v_shift))
        newi = jnp.where(krows < pos, iv,
                         jnp.where(krows == pos, gidx, i_shift))
        v_ref[...] = jnp.where(upd, newv, v)
        i_ref[...] = jnp.where(upd, newi, iv)
        return jnp.max(s_next, axis=0, keepdims=True)

    def cond(carry):
        return carry[0]

    def body(carry):
        _, m = carry
        m2 = _extract_one(m)
        cont = jnp.any(m2 > v_ref[_TOPK - 1:_TOPK, :])
        return cont, m2

    m0 = jnp.max(s_ref[...], axis=0, keepdims=True)
    cont0 = jnp.any(m0 > v_ref[_TOPK - 1:_TOPK, :])
    jax.lax.while_loop(cond, body, (cont0, m0))

    @pl.when(blk == nblk - 1)
    def _emit():
        vals_ref[...] = v_ref[:_TOPK, :]
        idx_ref[...] = i_ref[:_TOPK, :]


def kernel(queries, keys, k):
    del k
    q, d = queries.shape
    nkeys = keys.shape[0]
    bk = 1024
    nblk = (nkeys + bk - 1) // bk
    kpad = nblk * bk

    norms_text = jnp.linalg.norm(keys, axis=-1)
    norm_question = jnp.linalg.norm(queries, axis=-1)

    qt = queries.T
    keys_pad = jnp.pad(keys, ((0, kpad - nkeys), (0, 0)))
    nk = jnp.pad(norms_text, (0, kpad - nkeys))[:, None]
    nq = norm_question[None, :]

    body = functools.partial(_fused_topk_kernel, nkeys, bk, nblk)
    vals_t, idx_t = pl.pallas_call(
        body,
        grid=(nblk,),
        in_specs=[
            pl.BlockSpec((d, q), lambda i: (0, 0)),
            pl.BlockSpec((1, q), lambda i: (0, 0)),
            pl.BlockSpec((bk, d), lambda i: (i, 0)),
            pl.BlockSpec((bk, 1), lambda i: (i, 0)),
        ],
        out_specs=[
            pl.BlockSpec((_TOPK, q), lambda i: (0, 0)),
            pl.BlockSpec((_TOPK, q), lambda i: (0, 0)),
        ],
        out_shape=[
            jax.ShapeDtypeStruct((_TOPK, q), jnp.float32),
            jax.ShapeDtypeStruct((_TOPK, q), jnp.int32),
        ],
        scratch_shapes=[
            pltpu.VMEM((bk, q), jnp.float32),
            pltpu.VMEM((_KCAP, q), jnp.float32),
            pltpu.VMEM((_KCAP, q), jnp.int32),
        ],
        compiler_params=pltpu.CompilerParams(
            dimension_semantics=("arbitrary",)),
    )(qt, nq, keys_pad, nk)

    return vals_t.T, idx_t.T
